# baseline (speedup 1.0000x reference)
import functools

import jax
import jax.numpy as jnp
from jax import lax
from jax.experimental import pallas as pl
from jax.experimental.pallas import tpu as pltpu
from jax.experimental.pallas import tpu_sc as plsc

EMBED = 64


@functools.cache
def _make_gather(B: int, S: int):
    info = plsc.get_sparse_core_info()
    NC, NS = info.num_cores, info.num_subcores
    NW = NC * NS
    n_rows = B * S
    b_per_w = n_rows // NW
    C = 64
    NBUF = 4
    n_chunks = b_per_w // C
    n_groups = n_chunks // NBUF
    assert n_chunks % NBUF == 0 and S % C == 0
    mesh = plsc.VectorSubcoreMesh(core_axis_name="c", subcore_axis_name="s")

    scratch = [
        pltpu.VMEM((NBUF, C), jnp.int32),
        pltpu.VMEM((NBUF, C, 2 * EMBED), jnp.float32),
        pltpu.VMEM((NBUF, C, EMBED), jnp.float32),
    ] + [pltpu.SemaphoreType.DMA] * (3 * NBUF)

    @functools.partial(
        pl.kernel,
        mesh=mesh,
        out_type=jax.ShapeDtypeStruct((B, S, EMBED), jnp.float32),
        scratch_types=scratch,
    )
    def gather_kernel(idx_hbm, table_hbm, out_hbm, idx_v, rows_v, out_v, *sems):
        isem = sems[:NBUF]
        gsem = sems[NBUF:2 * NBUF]
        osem = sems[2 * NBUF:]
        wid = lax.axis_index("s") * NC + lax.axis_index("c")
        base = wid * b_per_w
        halves = S // C

        def start_idx(g, b):
            off = base + (g * NBUF + b) * C
            pltpu.async_copy(idx_hbm.at[pl.ds(off, C)], idx_v.at[b], isem[b])

        def wait_idx(b):
            pltpu.make_async_copy(
                idx_hbm.at[pl.ds(0, C)], idx_v.at[b], isem[b]).wait()

        def start_gather(b):
            pltpu.async_copy(table_hbm.at[idx_v.at[b]], rows_v.at[b], gsem[b])

        def wait_gather(b):
            pltpu.make_async_copy(
                table_hbm.at[idx_v.at[b]], rows_v.at[b], gsem[b]).wait()

        def compact(b):
            src = rows_v.at[b]
            dst = out_v.at[b]

            def row(r, carry):
                for k in range(EMBED // 16):
                    dst[r, pl.ds(16 * k, 16)] = src[r, pl.ds(16 * k, 16)]
                return carry

            lax.fori_loop(0, C, row, 0, unroll=4)

        def start_out(g, b):
            ci = wid * n_chunks + g * NBUF + b
            slab = ci // halves
            r0 = (ci % halves) * C
            pltpu.async_copy(
                out_v.at[b], out_hbm.at[slab, pl.ds(r0, C)], osem[b])

        def wait_out(b):
            pltpu.make_async_copy(
                out_v.at[b], out_hbm.at[0, pl.ds(0, C)], osem[b]).wait()

        for b in range(NBUF):
            start_idx(0, b)

        def group(g, carry):
            for b in range(NBUF):
                wait_idx(b)
                start_gather(b)
            for b in range(NBUF):
                wait_gather(b)

                @pl.when(g > 0)
                def _():
                    wait_out(b)

                compact(b)
                start_out(g, b)

                @pl.when(g < n_groups - 1)
                def _():
                    start_idx(g + 1, b)

            return carry

        lax.fori_loop(0, n_groups, group, 0)
        for b in range(NBUF):
            wait_out(b)

    return gather_kernel


def kernel(input_ids, attention_mask, table):
    B, S = input_ids.shape
    ids_flat = input_ids.reshape(B * S).astype(jnp.int32)
    table128 = jnp.concatenate([table, jnp.zeros_like(table)], axis=1)
    out = _make_gather(B, S)(ids_flat, table128)
    return out, attention_mask
